# Initial kernel scaffold; baseline (speedup 1.0000x reference)
import jax
import jax.numpy as jnp
from jax.experimental import pallas as pl


def kernel(h, edge_index0, edge_index1, W0, al0, ar0, W1, al1, ar1, Ws1, bs1, Ws2):
    raise NotImplementedError("write your pallas kernel here")


# speedup vs baseline: 15.8792x; 15.8792x over previous
import functools

import jax
import jax.numpy as jnp
from jax import lax
from jax.experimental import pallas as pl
from jax.experimental.pallas import tpu as pltpu
from jax.experimental.pallas import tpu_sc as plsc

N_NODES = 10000
N_EDGES = 320000
IN_SIZE = 128
OUT_SIZE = 64
HEADS = 8
D = OUT_SIZE * HEADS
HIDDEN = 64

NC = 2
NS = 16
L = 16
C = 128
NCHUNKS = N_EDGES // C
NPT = N_NODES // NS
BN = 400


def _bcast_lane(v, lane):
  idx = jnp.full((L, 1), lane, dtype=jnp.int32)
  dnums = lax.GatherDimensionNumbers(
      offset_dims=(), collapsed_slice_dims=(0,), start_index_map=(0,))
  return lax.gather(v, idx, dnums, (1,),
                    mode=lax.GatherScatterMode.PROMISE_IN_BOUNDS)


def _k0_body(h_ref, w_ref, alr_ref, wh0_ref, wh1_ref, elr_ref):
  wh = jnp.dot(h_ref[...], w_ref[...], preferred_element_type=jnp.float32)
  wh0_ref[...] = wh[:, :D]
  wh1_ref[...] = wh[:, D:]
  elr_ref[...] = jnp.dot(wh, alr_ref[...], preferred_element_type=jnp.float32)


def _k2a_body(z0_ref, z1_ref, ws1_ref, b_ref, w2_ref, out_ref):
  @pl.when(pl.program_id(0) == 0)
  def _():
    out_ref[...] = jnp.zeros_like(out_ref)

  t0 = jnp.tanh(jnp.dot(z0_ref[...], ws1_ref[...],
                        preferred_element_type=jnp.float32) + b_ref[...])
  t1 = jnp.tanh(jnp.dot(z1_ref[...], ws1_ref[...],
                        preferred_element_type=jnp.float32) + b_ref[...])
  q0 = jnp.sum(t0 * w2_ref[...]) * (1.0 / N_NODES)
  q1 = jnp.sum(t1 * w2_ref[...]) * (1.0 / N_NODES)
  r = lax.broadcasted_iota(jnp.int32, (8, 128), 0)
  c = lax.broadcasted_iota(jnp.int32, (8, 128), 1)
  out_ref[...] += (jnp.where((r == 0) & (c == 0), q0, 0.0)
                   + jnp.where((r == 0) & (c == 1), q1, 0.0))


def _k2b_body(sc_ref, z0_ref, z1_ref, out_ref):
  s0 = sc_ref[0]
  s1 = sc_ref[1]
  m = jnp.maximum(s0, s1)
  e0 = jnp.exp(s0 - m)
  e1 = jnp.exp(s1 - m)
  b0 = e0 / (e0 + e1)
  b1 = e1 / (e0 + e1)
  out_ref[...] = z0_ref[...] * b0 + z1_ref[...] * b1


def _sc_body(src0, dst0, src1, dst1, el0, er0, el1, er1, wh0r, wh1r,
             z0, z1,
             dAcc, oA, oB, sidx, didx, gidx, elb, erb, sb, wbA, wbB,
             zb2, dvb, sem):
  c = lax.axis_index("c")
  s = lax.axis_index("s")
  hbase = 4 * c
  nck = 156 + jnp.where(s < 4, 1, 0)
  nnb = 39 + jnp.where(s < 1, 1, 0)

  for mp in range(2):
    srcR, dstR, elR, erR, whR, zR = (
        (src0, dst0, el0, er0, wh0r, z0) if mp == 0 else
        (src1, dst1, el1, er1, wh1r, z1))
    for hp in range(2):
      hA = hbase + 2 * hp

      def _zclear_body(r_, _):
        zb2[r_, pl.ds(0, L)] = jnp.zeros((L,), jnp.float32)
        zb2[r_, pl.ds(L, L)] = jnp.zeros((L,), jnp.float32)
        zb2[r_, pl.ds(2 * L, L)] = jnp.zeros((L,), jnp.float32)
        zb2[r_, pl.ds(3 * L, L)] = jnp.zeros((L,), jnp.float32)
        zb2[r_, pl.ds(4 * L, L)] = jnp.zeros((L,), jnp.float32)
        zb2[r_, pl.ds(5 * L, L)] = jnp.zeros((L,), jnp.float32)
        zb2[r_, pl.ds(6 * L, L)] = jnp.zeros((L,), jnp.float32)
        zb2[r_, pl.ds(7 * L, L)] = jnp.zeros((L,), jnp.float32)
        dvb[r_, pl.ds(0, L)] = jnp.zeros((L,), jnp.float32)
        return _
      lax.fori_loop(0, 16, _zclear_body, 0)

      def _zero_body(k, _):
        r0 = (s + NS * k) * 16
        pltpu.sync_copy(zb2.at[pl.ds(0, 16), pl.ds(0, OUT_SIZE)],
                        oA.at[pl.ds(r0, 16)])
        pltpu.sync_copy(zb2.at[pl.ds(0, 16), pl.ds(0, OUT_SIZE)],
                        oB.at[pl.ds(r0, 16)])
        pltpu.sync_copy(dvb, dAcc.at[pl.ds(r0, 16)])
        return _
      lax.fori_loop(0, nnb, _zero_body, 0)
      plsc.subcore_barrier()

      def _chunk_body(k, _):
        base = (s + k * NS) * C
        pltpu.sync_copy(srcR.at[pl.ds(base, C)], sidx)
        pltpu.sync_copy(dstR.at[pl.ds(base, C)], didx)
        pltpu.async_copy(elR.at[sidx], elb, sem).wait()
        pltpu.async_copy(erR.at[didx], erb, sem).wait()

        def _s_body(i, _):
          v = elb[i] + erb[i]
          v = jnp.where(v >= 0.0, v, 0.2 * v)
          sb[i] = jnp.exp(v)
          return _
        lax.fori_loop(0, C, _s_body, 0)
        pltpu.sync_copy(sb, dAcc.at[didx], add=True)

        for j in range(2):
          h = hA + j
          wb = wbA if j == 0 else wbB
          oX = oA if j == 0 else oB

          def _gidx_body(i, _):
            gidx[pl.ds(i * L, L)] = sidx[pl.ds(i * L, L)] * 8 + h
            return _
          lax.fori_loop(0, C // L, _gidx_body, 0)
          pltpu.async_copy(whR.at[gidx], wb, sem).wait()

          def _scale_body(i, _):
            f = _bcast_lane(sb[i], h)
            wb[i, pl.ds(0, L)] = wb[i, pl.ds(0, L)] * f
            wb[i, pl.ds(L, L)] = wb[i, pl.ds(L, L)] * f
            wb[i, pl.ds(2 * L, L)] = wb[i, pl.ds(2 * L, L)] * f
            wb[i, pl.ds(3 * L, L)] = wb[i, pl.ds(3 * L, L)] * f
            return _
          lax.fori_loop(0, C, _scale_body, 0)
          pltpu.sync_copy(wb, oX.at[didx], add=True)
        return _
      lax.fori_loop(0, nck, _chunk_body, 0)
      plsc.subcore_barrier()

      def _div_blk(k, _):
        r0 = (s + NS * k) * 16
        pltpu.sync_copy(dAcc.at[pl.ds(r0, 16)], dvb)
        pltpu.sync_copy(oA.at[pl.ds(r0, 16)],
                        zb2.at[pl.ds(0, 16), pl.ds(0, OUT_SIZE)])
        pltpu.sync_copy(oB.at[pl.ds(r0, 16)],
                        zb2.at[pl.ds(0, 16), pl.ds(OUT_SIZE, OUT_SIZE)])

        def _div_row(r_, _):
          fA = _bcast_lane(dvb[r_], hA) + 1e-9
          fB = _bcast_lane(dvb[r_], hA + 1) + 1e-9
          zb2[r_, pl.ds(0, L)] = zb2[r_, pl.ds(0, L)] / fA
          zb2[r_, pl.ds(L, L)] = zb2[r_, pl.ds(L, L)] / fA
          zb2[r_, pl.ds(2 * L, L)] = zb2[r_, pl.ds(2 * L, L)] / fA
          zb2[r_, pl.ds(3 * L, L)] = zb2[r_, pl.ds(3 * L, L)] / fA
          zb2[r_, pl.ds(4 * L, L)] = zb2[r_, pl.ds(4 * L, L)] / fB
          zb2[r_, pl.ds(5 * L, L)] = zb2[r_, pl.ds(5 * L, L)] / fB
          zb2[r_, pl.ds(6 * L, L)] = zb2[r_, pl.ds(6 * L, L)] / fB
          zb2[r_, pl.ds(7 * L, L)] = zb2[r_, pl.ds(7 * L, L)] / fB
          return _
        lax.fori_loop(0, 16, _div_row, 0)
        pltpu.sync_copy(zb2, zR.at[pl.ds(r0, 16),
                                   pl.ds(hA * OUT_SIZE, 2 * OUT_SIZE)])
        return _
      lax.fori_loop(0, nnb, _div_blk, 0)
      plsc.subcore_barrier()


_sc_kernel = functools.partial(
    pl.kernel,
    out_type=[jax.ShapeDtypeStruct((N_NODES, D), jnp.float32),
              jax.ShapeDtypeStruct((N_NODES, D), jnp.float32)],
    mesh=plsc.VectorSubcoreMesh(core_axis_name="c", subcore_axis_name="s"),
    compiler_params=pltpu.CompilerParams(use_tc_tiling_on_sc=False),
    scratch_types=[
        pltpu.VMEM_SHARED((N_NODES, L), jnp.float32),
        pltpu.VMEM_SHARED((N_NODES, OUT_SIZE), jnp.float32),
        pltpu.VMEM_SHARED((N_NODES, OUT_SIZE), jnp.float32),
        pltpu.VMEM((C,), jnp.int32),
        pltpu.VMEM((C,), jnp.int32),
        pltpu.VMEM((C,), jnp.int32),
        pltpu.VMEM((C, L), jnp.float32),
        pltpu.VMEM((C, L), jnp.float32),
        pltpu.VMEM((C, L), jnp.float32),
        pltpu.VMEM((C, OUT_SIZE), jnp.float32),
        pltpu.VMEM((C, OUT_SIZE), jnp.float32),
        pltpu.VMEM((16, 2 * OUT_SIZE), jnp.float32),
        pltpu.VMEM((16, L), jnp.float32),
        pltpu.SemaphoreType.DMA,
    ])(_sc_body)


def _dup_logit_block(a):
  eye = jnp.eye(HEADS, dtype=jnp.float32)
  return jnp.einsum("hi,hj->hij", a, eye).reshape(D, HEADS)


@jax.jit
def kernel(h, edge_index0, edge_index1, W0, al0, ar0, W1, al1, ar1,
           Ws1, bs1, Ws2):
  Wcat = jnp.concatenate([W0, W1], axis=1)
  Bl0 = _dup_logit_block(al0)
  Br0 = _dup_logit_block(ar0)
  Bl1 = _dup_logit_block(al1)
  Br1 = _dup_logit_block(ar1)
  Z = jnp.zeros((D, 32), jnp.float32)
  ALR = jnp.concatenate([
      jnp.concatenate([Bl0, Bl0, Br0, Br0, Z], axis=1),
      jnp.concatenate([Z, Bl1, Bl1, Br1, Br1], axis=1),
  ], axis=0)

  wh0, wh1, elr = pl.pallas_call(
      _k0_body,
      grid=(N_NODES // BN,),
      in_specs=[
          pl.BlockSpec((BN, IN_SIZE), lambda i: (i, 0)),
          pl.BlockSpec((IN_SIZE, 2 * D), lambda i: (0, 0)),
          pl.BlockSpec((2 * D, 64), lambda i: (0, 0)),
      ],
      out_specs=[
          pl.BlockSpec((BN, D), lambda i: (i, 0)),
          pl.BlockSpec((BN, D), lambda i: (i, 0)),
          pl.BlockSpec((BN, 64), lambda i: (i, 0)),
      ],
      out_shape=[
          jax.ShapeDtypeStruct((N_NODES, D), jnp.float32),
          jax.ShapeDtypeStruct((N_NODES, D), jnp.float32),
          jax.ShapeDtypeStruct((N_NODES, 64), jnp.float32),
      ],
  )(h, Wcat, ALR)

  el0 = elr[:, 0:16]
  er0 = elr[:, 16:32]
  el1 = elr[:, 32:48]
  er1 = elr[:, 48:64]
  wh0r = wh0.reshape(N_NODES * HEADS, OUT_SIZE)
  wh1r = wh1.reshape(N_NODES * HEADS, OUT_SIZE)

  z0, z1 = _sc_kernel(
      edge_index0[0], edge_index0[1], edge_index1[0], edge_index1[1],
      el0, er0, el1, er1, wh0r, wh1r)

  scores_tile = pl.pallas_call(
      _k2a_body,
      grid=(N_NODES // BN,),
      in_specs=[
          pl.BlockSpec((BN, D), lambda i: (i, 0)),
          pl.BlockSpec((BN, D), lambda i: (i, 0)),
          pl.BlockSpec((D, HIDDEN), lambda i: (0, 0)),
          pl.BlockSpec((1, HIDDEN), lambda i: (0, 0)),
          pl.BlockSpec((1, HIDDEN), lambda i: (0, 0)),
      ],
      out_specs=pl.BlockSpec((8, 128), lambda i: (0, 0)),
      out_shape=jax.ShapeDtypeStruct((8, 128), jnp.float32),
  )(z0, z1, Ws1, bs1.reshape(1, HIDDEN), Ws2.reshape(1, HIDDEN))

  scores = scores_tile[0, :2]

  out = pl.pallas_call(
      _k2b_body,
      grid=(N_NODES // BN,),
      in_specs=[
          pl.BlockSpec(memory_space=pltpu.SMEM),
          pl.BlockSpec((BN, D), lambda i: (i, 0)),
          pl.BlockSpec((BN, D), lambda i: (i, 0)),
      ],
      out_specs=pl.BlockSpec((BN, D), lambda i: (i, 0)),
      out_shape=jax.ShapeDtypeStruct((N_NODES, D), jnp.float32),
  )(scores, z0, z1)
  return out


# speedup vs baseline: 18.5350x; 1.1672x over previous
import functools

import jax
import jax.numpy as jnp
from jax import lax
from jax.experimental import pallas as pl
from jax.experimental.pallas import tpu as pltpu
from jax.experimental.pallas import tpu_sc as plsc

N_NODES = 10000
N_EDGES = 320000
IN_SIZE = 128
OUT_SIZE = 64
HEADS = 8
D = OUT_SIZE * HEADS
HIDDEN = 64

NC = 2
NS = 16
L = 16
C = 128
NCHUNKS = N_EDGES // C
NPT = N_NODES // NS
BN = 400


def _bcast_lane(v, lane):
  idx = jnp.full((L, 1), lane, dtype=jnp.int32)
  dnums = lax.GatherDimensionNumbers(
      offset_dims=(), collapsed_slice_dims=(0,), start_index_map=(0,))
  return lax.gather(v, idx, dnums, (1,),
                    mode=lax.GatherScatterMode.PROMISE_IN_BOUNDS)


def _k0_body(h_ref, w_ref, alr_ref, wh0_ref, wh1_ref, elr_ref):
  wh = jnp.dot(h_ref[...], w_ref[...], preferred_element_type=jnp.float32)
  wh0_ref[...] = wh[:, :D]
  wh1_ref[...] = wh[:, D:]
  elr_ref[...] = jnp.dot(wh, alr_ref[...], preferred_element_type=jnp.float32)


def _k2a_body(z0_ref, z1_ref, ws1_ref, b_ref, w2_ref, out_ref):
  @pl.when(pl.program_id(0) == 0)
  def _():
    out_ref[...] = jnp.zeros_like(out_ref)

  t0 = jnp.tanh(jnp.dot(z0_ref[...], ws1_ref[...],
                        preferred_element_type=jnp.float32) + b_ref[...])
  t1 = jnp.tanh(jnp.dot(z1_ref[...], ws1_ref[...],
                        preferred_element_type=jnp.float32) + b_ref[...])
  q0 = jnp.sum(t0 * w2_ref[...]) * (1.0 / N_NODES)
  q1 = jnp.sum(t1 * w2_ref[...]) * (1.0 / N_NODES)
  r = lax.broadcasted_iota(jnp.int32, (8, 128), 0)
  c = lax.broadcasted_iota(jnp.int32, (8, 128), 1)
  out_ref[...] += (jnp.where((r == 0) & (c == 0), q0, 0.0)
                   + jnp.where((r == 0) & (c == 1), q1, 0.0))


def _k2b_body(sc_ref, z0_ref, z1_ref, out_ref):
  s0 = sc_ref[0]
  s1 = sc_ref[1]
  m = jnp.maximum(s0, s1)
  e0 = jnp.exp(s0 - m)
  e1 = jnp.exp(s1 - m)
  b0 = e0 / (e0 + e1)
  b1 = e1 / (e0 + e1)
  out_ref[...] = z0_ref[...] * b0 + z1_ref[...] * b1


def _sc_body(src0, dst0, src1, dst1, el0, er0, el1, er1, wh0r, wh1r,
             z0, z1,
             dAcc, oA, sidx, didx, gidx, elb, erb, sb, wb,
             zb2, dvb, sem):
  c = lax.axis_index("c")
  s = lax.axis_index("s")
  hbase = 4 * c
  nck = 156 + jnp.where(s < 4, 1, 0)
  nnb = 39 + jnp.where(s < 1, 1, 0)

  for mp in range(2):
    srcR, dstR, elR, erR, whR, zR = (
        (src0, dst0, el0, er0, wh0r, z0) if mp == 0 else
        (src1, dst1, el1, er1, wh1r, z1))
    for hp in range(2):
      hA = hbase + 2 * hp
      hpair = 2 * c + hp

      def _zclear_body(r_, _):
        zb2[r_, pl.ds(0, L)] = jnp.zeros((L,), jnp.float32)
        zb2[r_, pl.ds(L, L)] = jnp.zeros((L,), jnp.float32)
        zb2[r_, pl.ds(2 * L, L)] = jnp.zeros((L,), jnp.float32)
        zb2[r_, pl.ds(3 * L, L)] = jnp.zeros((L,), jnp.float32)
        zb2[r_, pl.ds(4 * L, L)] = jnp.zeros((L,), jnp.float32)
        zb2[r_, pl.ds(5 * L, L)] = jnp.zeros((L,), jnp.float32)
        zb2[r_, pl.ds(6 * L, L)] = jnp.zeros((L,), jnp.float32)
        zb2[r_, pl.ds(7 * L, L)] = jnp.zeros((L,), jnp.float32)
        dvb[r_, pl.ds(0, L)] = jnp.zeros((L,), jnp.float32)
        return _
      lax.fori_loop(0, 16, _zclear_body, 0)

      def _zero_body(k, _):
        r0 = (s + NS * k) * 16
        pltpu.sync_copy(zb2, oA.at[pl.ds(r0, 16)])
        pltpu.sync_copy(dvb, dAcc.at[pl.ds(r0, 16)])
        return _
      lax.fori_loop(0, nnb, _zero_body, 0)
      plsc.subcore_barrier()

      def _chunk_body(k, _):
        base = (s + k * NS) * C
        pltpu.sync_copy(srcR.at[pl.ds(base, C)], sidx)
        pltpu.sync_copy(dstR.at[pl.ds(base, C)], didx)
        pltpu.async_copy(elR.at[sidx], elb, sem).wait()
        pltpu.async_copy(erR.at[didx], erb, sem).wait()

        def _gidx_body(i, _):
          gidx[pl.ds(i * L, L)] = sidx[pl.ds(i * L, L)] * 4 + hpair
          return _
        lax.fori_loop(0, C // L, _gidx_body, 0)
        pltpu.async_copy(whR.at[gidx], wb, sem).wait()

        def _s_body(i, _):
          v = elb[i] + erb[i]
          v = jnp.where(v >= 0.0, v, 0.2 * v)
          sb[i] = jnp.exp(v)
          return _
        lax.fori_loop(0, C, _s_body, 0)
        pltpu.sync_copy(sb, dAcc.at[didx], add=True)

        def _scale_body(i, _):
          srow = sb[i]
          fA = _bcast_lane(srow, hA)
          fB = _bcast_lane(srow, hA + 1)
          wb[i, pl.ds(0, L)] = wb[i, pl.ds(0, L)] * fA
          wb[i, pl.ds(L, L)] = wb[i, pl.ds(L, L)] * fA
          wb[i, pl.ds(2 * L, L)] = wb[i, pl.ds(2 * L, L)] * fA
          wb[i, pl.ds(3 * L, L)] = wb[i, pl.ds(3 * L, L)] * fA
          wb[i, pl.ds(4 * L, L)] = wb[i, pl.ds(4 * L, L)] * fB
          wb[i, pl.ds(5 * L, L)] = wb[i, pl.ds(5 * L, L)] * fB
          wb[i, pl.ds(6 * L, L)] = wb[i, pl.ds(6 * L, L)] * fB
          wb[i, pl.ds(7 * L, L)] = wb[i, pl.ds(7 * L, L)] * fB
          return _
        lax.fori_loop(0, C, _scale_body, 0)
        pltpu.sync_copy(wb, oA.at[didx], add=True)
        return _
      lax.fori_loop(0, nck, _chunk_body, 0)
      plsc.subcore_barrier()

      def _div_blk(k, _):
        r0 = (s + NS * k) * 16
        pltpu.sync_copy(dAcc.at[pl.ds(r0, 16)], dvb)
        pltpu.sync_copy(oA.at[pl.ds(r0, 16)], zb2)

        def _div_row(r_, _):
          fA = _bcast_lane(dvb[r_], hA) + 1e-9
          fB = _bcast_lane(dvb[r_], hA + 1) + 1e-9
          zb2[r_, pl.ds(0, L)] = zb2[r_, pl.ds(0, L)] / fA
          zb2[r_, pl.ds(L, L)] = zb2[r_, pl.ds(L, L)] / fA
          zb2[r_, pl.ds(2 * L, L)] = zb2[r_, pl.ds(2 * L, L)] / fA
          zb2[r_, pl.ds(3 * L, L)] = zb2[r_, pl.ds(3 * L, L)] / fA
          zb2[r_, pl.ds(4 * L, L)] = zb2[r_, pl.ds(4 * L, L)] / fB
          zb2[r_, pl.ds(5 * L, L)] = zb2[r_, pl.ds(5 * L, L)] / fB
          zb2[r_, pl.ds(6 * L, L)] = zb2[r_, pl.ds(6 * L, L)] / fB
          zb2[r_, pl.ds(7 * L, L)] = zb2[r_, pl.ds(7 * L, L)] / fB
          return _
        lax.fori_loop(0, 16, _div_row, 0)
        pltpu.sync_copy(zb2, zR.at[pl.ds(r0, 16),
                                   pl.ds(hA * OUT_SIZE, 2 * OUT_SIZE)])
        return _
      lax.fori_loop(0, nnb, _div_blk, 0)
      plsc.subcore_barrier()


_sc_kernel = functools.partial(
    pl.kernel,
    out_type=[jax.ShapeDtypeStruct((N_NODES, D), jnp.float32),
              jax.ShapeDtypeStruct((N_NODES, D), jnp.float32)],
    mesh=plsc.VectorSubcoreMesh(core_axis_name="c", subcore_axis_name="s"),
    compiler_params=pltpu.CompilerParams(use_tc_tiling_on_sc=False),
    scratch_types=[
        pltpu.VMEM_SHARED((N_NODES, L), jnp.float32),
        pltpu.VMEM_SHARED((N_NODES, 2 * OUT_SIZE), jnp.float32),
        pltpu.VMEM((C,), jnp.int32),
        pltpu.VMEM((C,), jnp.int32),
        pltpu.VMEM((C,), jnp.int32),
        pltpu.VMEM((C, L), jnp.float32),
        pltpu.VMEM((C, L), jnp.float32),
        pltpu.VMEM((C, L), jnp.float32),
        pltpu.VMEM((C, 2 * OUT_SIZE), jnp.float32),
        pltpu.VMEM((16, 2 * OUT_SIZE), jnp.float32),
        pltpu.VMEM((16, L), jnp.float32),
        pltpu.SemaphoreType.DMA,
    ])(_sc_body)


def _dup_logit_block(a):
  eye = jnp.eye(HEADS, dtype=jnp.float32)
  return jnp.einsum("hi,hj->hij", a, eye).reshape(D, HEADS)


@jax.jit
def kernel(h, edge_index0, edge_index1, W0, al0, ar0, W1, al1, ar1,
           Ws1, bs1, Ws2):
  Wcat = jnp.concatenate([W0, W1], axis=1)
  Bl0 = _dup_logit_block(al0)
  Br0 = _dup_logit_block(ar0)
  Bl1 = _dup_logit_block(al1)
  Br1 = _dup_logit_block(ar1)
  Z = jnp.zeros((D, 32), jnp.float32)
  ALR = jnp.concatenate([
      jnp.concatenate([Bl0, Bl0, Br0, Br0, Z], axis=1),
      jnp.concatenate([Z, Bl1, Bl1, Br1, Br1], axis=1),
  ], axis=0)

  wh0, wh1, elr = pl.pallas_call(
      _k0_body,
      grid=(N_NODES // BN,),
      in_specs=[
          pl.BlockSpec((BN, IN_SIZE), lambda i: (i, 0)),
          pl.BlockSpec((IN_SIZE, 2 * D), lambda i: (0, 0)),
          pl.BlockSpec((2 * D, 64), lambda i: (0, 0)),
      ],
      out_specs=[
          pl.BlockSpec((BN, D), lambda i: (i, 0)),
          pl.BlockSpec((BN, D), lambda i: (i, 0)),
          pl.BlockSpec((BN, 64), lambda i: (i, 0)),
      ],
      out_shape=[
          jax.ShapeDtypeStruct((N_NODES, D), jnp.float32),
          jax.ShapeDtypeStruct((N_NODES, D), jnp.float32),
          jax.ShapeDtypeStruct((N_NODES, 64), jnp.float32),
      ],
  )(h, Wcat, ALR)

  el0 = elr[:, 0:16]
  er0 = elr[:, 16:32]
  el1 = elr[:, 32:48]
  er1 = elr[:, 48:64]
  wh0r = wh0.reshape(N_NODES * HEADS // 2, 2 * OUT_SIZE)
  wh1r = wh1.reshape(N_NODES * HEADS // 2, 2 * OUT_SIZE)

  z0, z1 = _sc_kernel(
      edge_index0[0], edge_index0[1], edge_index1[0], edge_index1[1],
      el0, er0, el1, er1, wh0r, wh1r)

  scores_tile = pl.pallas_call(
      _k2a_body,
      grid=(N_NODES // BN,),
      in_specs=[
          pl.BlockSpec((BN, D), lambda i: (i, 0)),
          pl.BlockSpec((BN, D), lambda i: (i, 0)),
          pl.BlockSpec((D, HIDDEN), lambda i: (0, 0)),
          pl.BlockSpec((1, HIDDEN), lambda i: (0, 0)),
          pl.BlockSpec((1, HIDDEN), lambda i: (0, 0)),
      ],
      out_specs=pl.BlockSpec((8, 128), lambda i: (0, 0)),
      out_shape=jax.ShapeDtypeStruct((8, 128), jnp.float32),
  )(z0, z1, Ws1, bs1.reshape(1, HIDDEN), Ws2.reshape(1, HIDDEN))

  scores = scores_tile[0, :2]

  out = pl.pallas_call(
      _k2b_body,
      grid=(N_NODES // BN,),
      in_specs=[
          pl.BlockSpec(memory_space=pltpu.SMEM),
          pl.BlockSpec((BN, D), lambda i: (i, 0)),
          pl.BlockSpec((BN, D), lambda i: (i, 0)),
      ],
      out_specs=pl.BlockSpec((BN, D), lambda i: (i, 0)),
      out_shape=jax.ShapeDtypeStruct((N_NODES, D), jnp.float32),
  )(scores, z0, z1)
  return out


# speedup vs baseline: 21.9443x; 1.1839x over previous
import functools

import jax
import jax.numpy as jnp
from jax import lax
from jax.experimental import pallas as pl
from jax.experimental.pallas import tpu as pltpu
from jax.experimental.pallas import tpu_sc as plsc

N_NODES = 10000
N_EDGES = 320000
IN_SIZE = 128
OUT_SIZE = 64
HEADS = 8
D = OUT_SIZE * HEADS
HIDDEN = 64

NC = 2
NS = 16
L = 16
C = 128
NCHUNKS = N_EDGES // C
NPT = N_NODES // NS
BN = 400


def _bcast_lane(v, lane):
  idx = jnp.full((L, 1), lane, dtype=jnp.int32)
  dnums = lax.GatherDimensionNumbers(
      offset_dims=(), collapsed_slice_dims=(0,), start_index_map=(0,))
  return lax.gather(v, idx, dnums, (1,),
                    mode=lax.GatherScatterMode.PROMISE_IN_BOUNDS)


def _k0_body(h_ref, w_ref, alr_ref, wh0_ref, wh1_ref, elr_ref):
  wh = jnp.dot(h_ref[...], w_ref[...], preferred_element_type=jnp.float32)
  wh0_ref[...] = wh[:, :D]
  wh1_ref[...] = wh[:, D:]
  elr_ref[...] = jnp.dot(wh, alr_ref[...], preferred_element_type=jnp.float32)


def _k2a_body(z0_ref, z1_ref, ws1_ref, b_ref, w2_ref, out_ref):
  @pl.when(pl.program_id(0) == 0)
  def _():
    out_ref[...] = jnp.zeros_like(out_ref)

  t0 = jnp.tanh(jnp.dot(z0_ref[...], ws1_ref[...],
                        preferred_element_type=jnp.float32) + b_ref[...])
  t1 = jnp.tanh(jnp.dot(z1_ref[...], ws1_ref[...],
                        preferred_element_type=jnp.float32) + b_ref[...])
  q0 = jnp.sum(t0 * w2_ref[...]) * (1.0 / N_NODES)
  q1 = jnp.sum(t1 * w2_ref[...]) * (1.0 / N_NODES)
  r = lax.broadcasted_iota(jnp.int32, (8, 128), 0)
  c = lax.broadcasted_iota(jnp.int32, (8, 128), 1)
  out_ref[...] += (jnp.where((r == 0) & (c == 0), q0, 0.0)
                   + jnp.where((r == 0) & (c == 1), q1, 0.0))


def _k2b_body(sc_ref, z0_ref, z1_ref, out_ref):
  s0 = sc_ref[0]
  s1 = sc_ref[1]
  m = jnp.maximum(s0, s1)
  e0 = jnp.exp(s0 - m)
  e1 = jnp.exp(s1 - m)
  b0 = e0 / (e0 + e1)
  b1 = e1 / (e0 + e1)
  out_ref[...] = z0_ref[...] * b0 + z1_ref[...] * b1


def _sc_body(src0, dst0, src1, dst1, el0, er0, el1, er1, wh0r, wh1r,
             z0, z1,
             dAcc, oA, sidx, didx, gidx, elb, erb, sb, wb,
             zb2, dvb, sem):
  c = lax.axis_index("c")
  s = lax.axis_index("s")
  hbase = 4 * c
  nck = 156 + jnp.where(s < 4, 1, 0)
  nnb = 39 + jnp.where(s < 1, 1, 0)

  for mp in range(2):
    srcR, dstR, elR, erR, whR, zR = (
        (src0, dst0, el0, er0, wh0r, z0) if mp == 0 else
        (src1, dst1, el1, er1, wh1r, z1))
    for hp in range(2):
      hA = hbase + 2 * hp
      hpair = 2 * c + hp

      @plsc.parallel_loop(0, 16, unroll=2)
      def _zclear_body(r_):
        zb2[r_, pl.ds(0, L)] = jnp.zeros((L,), jnp.float32)
        zb2[r_, pl.ds(L, L)] = jnp.zeros((L,), jnp.float32)
        zb2[r_, pl.ds(2 * L, L)] = jnp.zeros((L,), jnp.float32)
        zb2[r_, pl.ds(3 * L, L)] = jnp.zeros((L,), jnp.float32)
        zb2[r_, pl.ds(4 * L, L)] = jnp.zeros((L,), jnp.float32)
        zb2[r_, pl.ds(5 * L, L)] = jnp.zeros((L,), jnp.float32)
        zb2[r_, pl.ds(6 * L, L)] = jnp.zeros((L,), jnp.float32)
        zb2[r_, pl.ds(7 * L, L)] = jnp.zeros((L,), jnp.float32)
        dvb[r_, pl.ds(0, L)] = jnp.zeros((L,), jnp.float32)

      def _zero_body(k, _):
        r0 = (s + NS * k) * 16
        pltpu.sync_copy(zb2, oA.at[pl.ds(r0, 16)])
        pltpu.sync_copy(dvb, dAcc.at[pl.ds(r0, 16)])
        return _
      lax.fori_loop(0, nnb, _zero_body, 0)
      plsc.subcore_barrier()

      def _chunk_body(k, _):
        base = (s + k * NS) * C
        pltpu.sync_copy(srcR.at[pl.ds(base, C)], sidx)
        pltpu.sync_copy(dstR.at[pl.ds(base, C)], didx)
        pltpu.async_copy(elR.at[sidx], elb, sem).wait()
        pltpu.async_copy(erR.at[didx], erb, sem).wait()

        @plsc.parallel_loop(0, C // L, unroll=4)
        def _gidx_body(i):
          gidx[pl.ds(i * L, L)] = sidx[pl.ds(i * L, L)] * 4 + hpair
        pltpu.async_copy(whR.at[gidx], wb, sem).wait()

        @plsc.parallel_loop(0, C, unroll=4)
        def _s_body(i):
          v = elb[i] + erb[i]
          v = jnp.where(v >= 0.0, v, 0.2 * v)
          sb[i] = jnp.exp(v)
        pltpu.sync_copy(sb, dAcc.at[didx], add=True)

        @plsc.parallel_loop(0, C, unroll=2)
        def _scale_body(i):
          srow = sb[i]
          fA = _bcast_lane(srow, hA)
          fB = _bcast_lane(srow, hA + 1)
          wb[i, pl.ds(0, L)] = wb[i, pl.ds(0, L)] * fA
          wb[i, pl.ds(L, L)] = wb[i, pl.ds(L, L)] * fA
          wb[i, pl.ds(2 * L, L)] = wb[i, pl.ds(2 * L, L)] * fA
          wb[i, pl.ds(3 * L, L)] = wb[i, pl.ds(3 * L, L)] * fA
          wb[i, pl.ds(4 * L, L)] = wb[i, pl.ds(4 * L, L)] * fB
          wb[i, pl.ds(5 * L, L)] = wb[i, pl.ds(5 * L, L)] * fB
          wb[i, pl.ds(6 * L, L)] = wb[i, pl.ds(6 * L, L)] * fB
          wb[i, pl.ds(7 * L, L)] = wb[i, pl.ds(7 * L, L)] * fB
        pltpu.sync_copy(wb, oA.at[didx], add=True)
        return _
      lax.fori_loop(0, nck, _chunk_body, 0)
      plsc.subcore_barrier()

      def _div_blk(k, _):
        r0 = (s + NS * k) * 16
        pltpu.sync_copy(dAcc.at[pl.ds(r0, 16)], dvb)
        pltpu.sync_copy(oA.at[pl.ds(r0, 16)], zb2)

        @plsc.parallel_loop(0, 16, unroll=2)
        def _div_row(r_):
          fA = _bcast_lane(dvb[r_], hA) + 1e-9
          fB = _bcast_lane(dvb[r_], hA + 1) + 1e-9
          zb2[r_, pl.ds(0, L)] = zb2[r_, pl.ds(0, L)] / fA
          zb2[r_, pl.ds(L, L)] = zb2[r_, pl.ds(L, L)] / fA
          zb2[r_, pl.ds(2 * L, L)] = zb2[r_, pl.ds(2 * L, L)] / fA
          zb2[r_, pl.ds(3 * L, L)] = zb2[r_, pl.ds(3 * L, L)] / fA
          zb2[r_, pl.ds(4 * L, L)] = zb2[r_, pl.ds(4 * L, L)] / fB
          zb2[r_, pl.ds(5 * L, L)] = zb2[r_, pl.ds(5 * L, L)] / fB
          zb2[r_, pl.ds(6 * L, L)] = zb2[r_, pl.ds(6 * L, L)] / fB
          zb2[r_, pl.ds(7 * L, L)] = zb2[r_, pl.ds(7 * L, L)] / fB
        pltpu.sync_copy(zb2, zR.at[pl.ds(r0, 16),
                                   pl.ds(hA * OUT_SIZE, 2 * OUT_SIZE)])
        return _
      lax.fori_loop(0, nnb, _div_blk, 0)
      plsc.subcore_barrier()


_sc_kernel = functools.partial(
    pl.kernel,
    out_type=[jax.ShapeDtypeStruct((N_NODES, D), jnp.float32),
              jax.ShapeDtypeStruct((N_NODES, D), jnp.float32)],
    mesh=plsc.VectorSubcoreMesh(core_axis_name="c", subcore_axis_name="s"),
    compiler_params=pltpu.CompilerParams(use_tc_tiling_on_sc=False),
    scratch_types=[
        pltpu.VMEM_SHARED((N_NODES, L), jnp.float32),
        pltpu.VMEM_SHARED((N_NODES, 2 * OUT_SIZE), jnp.float32),
        pltpu.VMEM((C,), jnp.int32),
        pltpu.VMEM((C,), jnp.int32),
        pltpu.VMEM((C,), jnp.int32),
        pltpu.VMEM((C, L), jnp.float32),
        pltpu.VMEM((C, L), jnp.float32),
        pltpu.VMEM((C, L), jnp.float32),
        pltpu.VMEM((C, 2 * OUT_SIZE), jnp.float32),
        pltpu.VMEM((16, 2 * OUT_SIZE), jnp.float32),
        pltpu.VMEM((16, L), jnp.float32),
        pltpu.SemaphoreType.DMA,
    ])(_sc_body)


def _dup_logit_block(a):
  eye = jnp.eye(HEADS, dtype=jnp.float32)
  return jnp.einsum("hi,hj->hij", a, eye).reshape(D, HEADS)


@jax.jit
def kernel(h, edge_index0, edge_index1, W0, al0, ar0, W1, al1, ar1,
           Ws1, bs1, Ws2):
  Wcat = jnp.concatenate([W0, W1], axis=1)
  Bl0 = _dup_logit_block(al0)
  Br0 = _dup_logit_block(ar0)
  Bl1 = _dup_logit_block(al1)
  Br1 = _dup_logit_block(ar1)
  Z = jnp.zeros((D, 32), jnp.float32)
  ALR = jnp.concatenate([
      jnp.concatenate([Bl0, Bl0, Br0, Br0, Z], axis=1),
      jnp.concatenate([Z, Bl1, Bl1, Br1, Br1], axis=1),
  ], axis=0)

  wh0, wh1, elr = pl.pallas_call(
      _k0_body,
      grid=(N_NODES // BN,),
      in_specs=[
          pl.BlockSpec((BN, IN_SIZE), lambda i: (i, 0)),
          pl.BlockSpec((IN_SIZE, 2 * D), lambda i: (0, 0)),
          pl.BlockSpec((2 * D, 64), lambda i: (0, 0)),
      ],
      out_specs=[
          pl.BlockSpec((BN, D), lambda i: (i, 0)),
          pl.BlockSpec((BN, D), lambda i: (i, 0)),
          pl.BlockSpec((BN, 64), lambda i: (i, 0)),
      ],
      out_shape=[
          jax.ShapeDtypeStruct((N_NODES, D), jnp.float32),
          jax.ShapeDtypeStruct((N_NODES, D), jnp.float32),
          jax.ShapeDtypeStruct((N_NODES, 64), jnp.float32),
      ],
  )(h, Wcat, ALR)

  el0 = elr[:, 0:16]
  er0 = elr[:, 16:32]
  el1 = elr[:, 32:48]
  er1 = elr[:, 48:64]
  wh0r = wh0.reshape(N_NODES * HEADS // 2, 2 * OUT_SIZE)
  wh1r = wh1.reshape(N_NODES * HEADS // 2, 2 * OUT_SIZE)

  z0, z1 = _sc_kernel(
      edge_index0[0], edge_index0[1], edge_index1[0], edge_index1[1],
      el0, er0, el1, er1, wh0r, wh1r)

  scores_tile = pl.pallas_call(
      _k2a_body,
      grid=(N_NODES // BN,),
      in_specs=[
          pl.BlockSpec((BN, D), lambda i: (i, 0)),
          pl.BlockSpec((BN, D), lambda i: (i, 0)),
          pl.BlockSpec((D, HIDDEN), lambda i: (0, 0)),
          pl.BlockSpec((1, HIDDEN), lambda i: (0, 0)),
          pl.BlockSpec((1, HIDDEN), lambda i: (0, 0)),
      ],
      out_specs=pl.BlockSpec((8, 128), lambda i: (0, 0)),
      out_shape=jax.ShapeDtypeStruct((8, 128), jnp.float32),
  )(z0, z1, Ws1, bs1.reshape(1, HIDDEN), Ws2.reshape(1, HIDDEN))

  scores = scores_tile[0, :2]

  out = pl.pallas_call(
      _k2b_body,
      grid=(N_NODES // BN,),
      in_specs=[
          pl.BlockSpec(memory_space=pltpu.SMEM),
          pl.BlockSpec((BN, D), lambda i: (i, 0)),
          pl.BlockSpec((BN, D), lambda i: (i, 0)),
      ],
      out_specs=pl.BlockSpec((BN, D), lambda i: (i, 0)),
      out_shape=jax.ShapeDtypeStruct((N_NODES, D), jnp.float32),
  )(scores, z0, z1)
  return out


# speedup vs baseline: 31.9821x; 1.4574x over previous
import functools

import jax
import jax.numpy as jnp
from jax import lax
from jax.experimental import pallas as pl
from jax.experimental.pallas import tpu as pltpu
from jax.experimental.pallas import tpu_sc as plsc

N_NODES = 10000
N_EDGES = 320000
IN_SIZE = 128
OUT_SIZE = 64
HEADS = 8
D = OUT_SIZE * HEADS
HIDDEN = 64

NC = 2
NS = 16
L = 16
C = 64
NCHUNKS = N_EDGES // C
NPT = N_NODES // NS
BN = 400


def _bcast_lane(v, lane):
  idx = jnp.full((L, 1), lane, dtype=jnp.int32)
  dnums = lax.GatherDimensionNumbers(
      offset_dims=(), collapsed_slice_dims=(0,), start_index_map=(0,))
  return lax.gather(v, idx, dnums, (1,),
                    mode=lax.GatherScatterMode.PROMISE_IN_BOUNDS)


def _k0_body(h_ref, w_ref, alr_ref, wh0_ref, wh1_ref, elr_ref):
  wh = jnp.dot(h_ref[...], w_ref[...], preferred_element_type=jnp.float32)
  wh0_ref[...] = wh[:, :D]
  wh1_ref[...] = wh[:, D:]
  elr_ref[...] = jnp.dot(wh, alr_ref[...], preferred_element_type=jnp.float32)


def _k2a_body(z0_ref, z1_ref, ws1_ref, b_ref, w2_ref, out_ref):
  @pl.when(pl.program_id(0) == 0)
  def _():
    out_ref[...] = jnp.zeros_like(out_ref)

  t0 = jnp.tanh(jnp.dot(z0_ref[...], ws1_ref[...],
                        preferred_element_type=jnp.float32) + b_ref[...])
  t1 = jnp.tanh(jnp.dot(z1_ref[...], ws1_ref[...],
                        preferred_element_type=jnp.float32) + b_ref[...])
  q0 = jnp.sum(t0 * w2_ref[...]) * (1.0 / N_NODES)
  q1 = jnp.sum(t1 * w2_ref[...]) * (1.0 / N_NODES)
  r = lax.broadcasted_iota(jnp.int32, (8, 128), 0)
  c = lax.broadcasted_iota(jnp.int32, (8, 128), 1)
  out_ref[...] += (jnp.where((r == 0) & (c == 0), q0, 0.0)
                   + jnp.where((r == 0) & (c == 1), q1, 0.0))


def _k2b_body(sc_ref, z0_ref, z1_ref, out_ref):
  s0 = sc_ref[0]
  s1 = sc_ref[1]
  m = jnp.maximum(s0, s1)
  e0 = jnp.exp(s0 - m)
  e1 = jnp.exp(s1 - m)
  b0 = e0 / (e0 + e1)
  b1 = e1 / (e0 + e1)
  out_ref[...] = z0_ref[...] * b0 + z1_ref[...] * b1


def _sc_body(src0, dst0, src1, dst1, el0, er0, el1, er1, wh0r, wh1r,
             z0, z1,
             dAcc, oA,
             sidx0, didx0, gidx0, elb0, erb0, wb0,
             sidx1, didx1, gidx1, elb1, erb1, wb1,
             sb, zb2, dvb, sem0, sem1):
  bufs0 = (sidx0, didx0, gidx0, elb0, erb0, wb0, sem0)
  bufs1 = (sidx1, didx1, gidx1, elb1, erb1, wb1, sem1)
  c = lax.axis_index("c")
  s = lax.axis_index("s")
  hbase = 4 * c
  nck = 312 + jnp.where(s < 8, 1, 0)
  nnb = 39 + jnp.where(s < 1, 1, 0)

  for mp in range(2):
    srcR, dstR, elR, erR, whR, zR = (
        (src0, dst0, el0, er0, wh0r, z0) if mp == 0 else
        (src1, dst1, el1, er1, wh1r, z1))
    for hp in range(2):
      hA = hbase + 2 * hp
      hpair = 2 * c + hp

      @plsc.parallel_loop(0, 16, unroll=2)
      def _zclear_body(r_):
        zb2[r_, pl.ds(0, L)] = jnp.zeros((L,), jnp.float32)
        zb2[r_, pl.ds(L, L)] = jnp.zeros((L,), jnp.float32)
        zb2[r_, pl.ds(2 * L, L)] = jnp.zeros((L,), jnp.float32)
        zb2[r_, pl.ds(3 * L, L)] = jnp.zeros((L,), jnp.float32)
        zb2[r_, pl.ds(4 * L, L)] = jnp.zeros((L,), jnp.float32)
        zb2[r_, pl.ds(5 * L, L)] = jnp.zeros((L,), jnp.float32)
        zb2[r_, pl.ds(6 * L, L)] = jnp.zeros((L,), jnp.float32)
        zb2[r_, pl.ds(7 * L, L)] = jnp.zeros((L,), jnp.float32)
        dvb[r_, pl.ds(0, L)] = jnp.zeros((L,), jnp.float32)

      def _zero_body(k, _):
        r0 = (s + NS * k) * 16
        pltpu.sync_copy(zb2, oA.at[pl.ds(r0, 16)])
        pltpu.sync_copy(dvb, dAcc.at[pl.ds(r0, 16)])
        return _
      lax.fori_loop(0, nnb, _zero_body, 0)
      plsc.subcore_barrier()

      def _prep_issue(k, bufs):
        sx, dx, gx, eb, rb, wbx, sm = bufs
        base = (s + k * NS) * C
        pltpu.sync_copy(srcR.at[pl.ds(base, C)], sx)
        pltpu.sync_copy(dstR.at[pl.ds(base, C)], dx)

        @plsc.parallel_loop(0, C // L, unroll=4)
        def _gidx_body(i):
          gx[pl.ds(i * L, L)] = sx[pl.ds(i * L, L)] * 4 + hpair
        pltpu.async_copy(elR.at[sx], eb, sm)
        pltpu.async_copy(erR.at[dx], rb, sm)
        pltpu.async_copy(whR.at[gx], wbx, sm)

      def _consume(bufs):
        sx, dx, gx, eb, rb, wbx, sm = bufs
        pltpu.make_async_copy(elR.at[sx], eb, sm).wait()
        pltpu.make_async_copy(erR.at[dx], rb, sm).wait()
        pltpu.make_async_copy(whR.at[gx], wbx, sm).wait()

        @plsc.parallel_loop(0, C, unroll=4)
        def _s_body(i):
          v = eb[i] + rb[i]
          v = jnp.where(v >= 0.0, v, 0.2 * v)
          sb[i] = jnp.exp(v)
        pltpu.sync_copy(sb, dAcc.at[dx], add=True)

        @plsc.parallel_loop(0, C, unroll=2)
        def _scale_body(i):
          srow = sb[i]
          fA = _bcast_lane(srow, hA)
          fB = _bcast_lane(srow, hA + 1)
          wbx[i, pl.ds(0, L)] = wbx[i, pl.ds(0, L)] * fA
          wbx[i, pl.ds(L, L)] = wbx[i, pl.ds(L, L)] * fA
          wbx[i, pl.ds(2 * L, L)] = wbx[i, pl.ds(2 * L, L)] * fA
          wbx[i, pl.ds(3 * L, L)] = wbx[i, pl.ds(3 * L, L)] * fA
          wbx[i, pl.ds(4 * L, L)] = wbx[i, pl.ds(4 * L, L)] * fB
          wbx[i, pl.ds(5 * L, L)] = wbx[i, pl.ds(5 * L, L)] * fB
          wbx[i, pl.ds(6 * L, L)] = wbx[i, pl.ds(6 * L, L)] * fB
          wbx[i, pl.ds(7 * L, L)] = wbx[i, pl.ds(7 * L, L)] * fB
        pltpu.sync_copy(wbx, oA.at[dx], add=True)

      _prep_issue(0, bufs0)

      def _pipe_body(k2, carry):
        _prep_issue(2 * k2 + 1, bufs1)
        _consume(bufs0)

        @pl.when(2 * k2 + 2 < nck)
        def _():
          _prep_issue(2 * k2 + 2, bufs0)
        _consume(bufs1)
        return carry
      lax.fori_loop(0, 156, _pipe_body, 0)

      @pl.when(s < 8)
      def _():
        _consume(bufs0)
      plsc.subcore_barrier()

      def _div_blk(k, _):
        r0 = (s + NS * k) * 16
        pltpu.sync_copy(dAcc.at[pl.ds(r0, 16)], dvb)
        pltpu.sync_copy(oA.at[pl.ds(r0, 16)], zb2)

        @plsc.parallel_loop(0, 16, unroll=2)
        def _div_row(r_):
          fA = _bcast_lane(dvb[r_], hA) + 1e-9
          fB = _bcast_lane(dvb[r_], hA + 1) + 1e-9
          zb2[r_, pl.ds(0, L)] = zb2[r_, pl.ds(0, L)] / fA
          zb2[r_, pl.ds(L, L)] = zb2[r_, pl.ds(L, L)] / fA
          zb2[r_, pl.ds(2 * L, L)] = zb2[r_, pl.ds(2 * L, L)] / fA
          zb2[r_, pl.ds(3 * L, L)] = zb2[r_, pl.ds(3 * L, L)] / fA
          zb2[r_, pl.ds(4 * L, L)] = zb2[r_, pl.ds(4 * L, L)] / fB
          zb2[r_, pl.ds(5 * L, L)] = zb2[r_, pl.ds(5 * L, L)] / fB
          zb2[r_, pl.ds(6 * L, L)] = zb2[r_, pl.ds(6 * L, L)] / fB
          zb2[r_, pl.ds(7 * L, L)] = zb2[r_, pl.ds(7 * L, L)] / fB
        pltpu.sync_copy(zb2, zR.at[pl.ds(r0, 16),
                                   pl.ds(hA * OUT_SIZE, 2 * OUT_SIZE)])
        return _
      lax.fori_loop(0, nnb, _div_blk, 0)
      plsc.subcore_barrier()


_sc_kernel = functools.partial(
    pl.kernel,
    out_type=[jax.ShapeDtypeStruct((N_NODES, D), jnp.float32),
              jax.ShapeDtypeStruct((N_NODES, D), jnp.float32)],
    mesh=plsc.VectorSubcoreMesh(core_axis_name="c", subcore_axis_name="s"),
    compiler_params=pltpu.CompilerParams(use_tc_tiling_on_sc=False),
    scratch_types=[
        pltpu.VMEM_SHARED((N_NODES, L), jnp.float32),
        pltpu.VMEM_SHARED((N_NODES, 2 * OUT_SIZE), jnp.float32),
        pltpu.VMEM((C,), jnp.int32),
        pltpu.VMEM((C,), jnp.int32),
        pltpu.VMEM((C,), jnp.int32),
        pltpu.VMEM((C, L), jnp.float32),
        pltpu.VMEM((C, L), jnp.float32),
        pltpu.VMEM((C, 2 * OUT_SIZE), jnp.float32),
        pltpu.VMEM((C,), jnp.int32),
        pltpu.VMEM((C,), jnp.int32),
        pltpu.VMEM((C,), jnp.int32),
        pltpu.VMEM((C, L), jnp.float32),
        pltpu.VMEM((C, L), jnp.float32),
        pltpu.VMEM((C, 2 * OUT_SIZE), jnp.float32),
        pltpu.VMEM((C, L), jnp.float32),
        pltpu.VMEM((16, 2 * OUT_SIZE), jnp.float32),
        pltpu.VMEM((16, L), jnp.float32),
        pltpu.SemaphoreType.DMA,
        pltpu.SemaphoreType.DMA,
    ])(_sc_body)


def _dup_logit_block(a):
  eye = jnp.eye(HEADS, dtype=jnp.float32)
  return jnp.einsum("hi,hj->hij", a, eye).reshape(D, HEADS)


@jax.jit
def kernel(h, edge_index0, edge_index1, W0, al0, ar0, W1, al1, ar1,
           Ws1, bs1, Ws2):
  Wcat = jnp.concatenate([W0, W1], axis=1)
  Bl0 = _dup_logit_block(al0)
  Br0 = _dup_logit_block(ar0)
  Bl1 = _dup_logit_block(al1)
  Br1 = _dup_logit_block(ar1)
  Z = jnp.zeros((D, 32), jnp.float32)
  ALR = jnp.concatenate([
      jnp.concatenate([Bl0, Bl0, Br0, Br0, Z], axis=1),
      jnp.concatenate([Z, Bl1, Bl1, Br1, Br1], axis=1),
  ], axis=0)

  wh0, wh1, elr = pl.pallas_call(
      _k0_body,
      grid=(N_NODES // BN,),
      in_specs=[
          pl.BlockSpec((BN, IN_SIZE), lambda i: (i, 0)),
          pl.BlockSpec((IN_SIZE, 2 * D), lambda i: (0, 0)),
          pl.BlockSpec((2 * D, 64), lambda i: (0, 0)),
      ],
      out_specs=[
          pl.BlockSpec((BN, D), lambda i: (i, 0)),
          pl.BlockSpec((BN, D), lambda i: (i, 0)),
          pl.BlockSpec((BN, 64), lambda i: (i, 0)),
      ],
      out_shape=[
          jax.ShapeDtypeStruct((N_NODES, D), jnp.float32),
          jax.ShapeDtypeStruct((N_NODES, D), jnp.float32),
          jax.ShapeDtypeStruct((N_NODES, 64), jnp.float32),
      ],
  )(h, Wcat, ALR)

  el0 = elr[:, 0:16]
  er0 = elr[:, 16:32]
  el1 = elr[:, 32:48]
  er1 = elr[:, 48:64]
  wh0r = wh0.reshape(N_NODES * HEADS // 2, 2 * OUT_SIZE)
  wh1r = wh1.reshape(N_NODES * HEADS // 2, 2 * OUT_SIZE)

  z0, z1 = _sc_kernel(
      edge_index0[0], edge_index0[1], edge_index1[0], edge_index1[1],
      el0, er0, el1, er1, wh0r, wh1r)

  scores_tile = pl.pallas_call(
      _k2a_body,
      grid=(N_NODES // BN,),
      in_specs=[
          pl.BlockSpec((BN, D), lambda i: (i, 0)),
          pl.BlockSpec((BN, D), lambda i: (i, 0)),
          pl.BlockSpec((D, HIDDEN), lambda i: (0, 0)),
          pl.BlockSpec((1, HIDDEN), lambda i: (0, 0)),
          pl.BlockSpec((1, HIDDEN), lambda i: (0, 0)),
      ],
      out_specs=pl.BlockSpec((8, 128), lambda i: (0, 0)),
      out_shape=jax.ShapeDtypeStruct((8, 128), jnp.float32),
  )(z0, z1, Ws1, bs1.reshape(1, HIDDEN), Ws2.reshape(1, HIDDEN))

  scores = scores_tile[0, :2]

  out = pl.pallas_call(
      _k2b_body,
      grid=(N_NODES // BN,),
      in_specs=[
          pl.BlockSpec(memory_space=pltpu.SMEM),
          pl.BlockSpec((BN, D), lambda i: (i, 0)),
          pl.BlockSpec((BN, D), lambda i: (i, 0)),
      ],
      out_specs=pl.BlockSpec((BN, D), lambda i: (i, 0)),
      out_shape=jax.ShapeDtypeStruct((N_NODES, D), jnp.float32),
  )(scores, z0, z1)
  return out
